# baseline (speedup 1.0000x reference)
import jax
import jax.numpy as jnp
from jax.experimental import pallas as pl
from jax.experimental.pallas import tpu as pltpu


_TB = 1024


def _body(x_ref, w1_ref, b1_ref, wr_ref, b2_ref, dec_ref, keep_ref):
    TB = x_ref.shape[0]
    S = wr_ref.shape[0]
    K = wr_ref.shape[2] // 2
    acc = jnp.zeros((TB, 2 * K), jnp.float32)
    for s in range(S):
        xs = x_ref[:, 128 * s : 128 * (s + 1)]
        h = jnp.maximum(
            jax.lax.dot_general(
                xs, w1_ref[...], (((1,), (0,)), ((), ())),
                preferred_element_type=jnp.float32,
            )
            + b1_ref[...],
            0.0,
        )
        acc = acc + jax.lax.dot_general(
            h, wr_ref[s], (((1,), (0,)), ((), ())),
            preferred_element_type=jnp.float32,
        )
    delta = (acc[:, K:] + b2_ref[0, 1]) - (acc[:, :K] + b2_ref[0, 0])
    rowmax = jnp.max(delta, axis=1, keepdims=True)
    lane = jax.lax.broadcasted_iota(jnp.int32, delta.shape, 1)
    dec = jnp.where((delta > 0.0) | ((lane == 0) & (rowmax <= 0.0)), 1.0, 0.0)
    dec_ref[...] = dec
    keep_ref[...] = jax.nn.sigmoid(delta)


def kernel(slots, W1, b1, W2, b2):
    B, K, D = slots.shape
    F = W1.shape[1]
    S = K // 2
    x = jnp.reshape(slots, (B, K * D))
    z = jnp.zeros((D, F), jnp.float32)
    w1bd = jnp.block([[W1, z], [z, W1]])
    b1bd = jnp.concatenate([b1, b1]).reshape(1, 2 * F)
    cols = jax.lax.broadcasted_iota(jnp.int32, (S, 2 * F, 2 * K), 2)
    rows = jax.lax.broadcasted_iota(jnp.int32, (S, 2 * F, 2 * K), 1)
    sidx = jax.lax.broadcasted_iota(jnp.int32, (S, 2 * F, 2 * K), 0)
    cmod = cols % K
    v0 = jnp.concatenate([W2[:, 0], W2[:, 0]]).reshape(1, 2 * F, 1)
    v1 = jnp.concatenate([W2[:, 1], W2[:, 1]]).reshape(1, 2 * F, 1)
    vals = jnp.where(cols < K, v0, v1)
    wr = jnp.where(
        (cmod == 2 * sidx + (rows >= F).astype(jnp.int32))
        & ((rows < F) == (cmod == 2 * sidx)),
        vals,
        0.0,
    ).astype(jnp.float32)
    b2r = b2.reshape(1, 2)

    grid = (B // _TB,)
    dec, keep = pl.pallas_call(
        _body,
        grid=grid,
        in_specs=[
            pl.BlockSpec((_TB, K * D), lambda i: (i, 0)),
            pl.BlockSpec((2 * D, 2 * F), lambda i: (0, 0)),
            pl.BlockSpec((1, 2 * F), lambda i: (0, 0)),
            pl.BlockSpec((S, 2 * F, 2 * K), lambda i: (0, 0, 0)),
            pl.BlockSpec(memory_space=pltpu.SMEM),
        ],
        out_specs=[
            pl.BlockSpec((_TB, K), lambda i: (i, 0)),
            pl.BlockSpec((_TB, K), lambda i: (i, 0)),
        ],
        out_shape=[
            jax.ShapeDtypeStruct((B, K), jnp.float32),
            jax.ShapeDtypeStruct((B, K), jnp.float32),
        ],
        compiler_params=pltpu.CompilerParams(
            dimension_semantics=("parallel",),
        ),
    )(x, w1bd, b1bd, wr, b2r)
    return (dec, keep)


# speedup vs baseline: 1.0580x; 1.0580x over previous
import jax
import jax.numpy as jnp
from jax.experimental import pallas as pl
from jax.experimental.pallas import tpu as pltpu


_NQ = 4
_TBQ = 256


def _make_body(nq):
    def _body(*refs):
        x_refs = refs[:nq]
        w1_ref, b1_ref, wr_ref, b2_ref, dec_ref, keep_ref = refs[nq:]
        S = wr_ref.shape[0]
        K = wr_ref.shape[2] // 2
        for q, x_ref in enumerate(x_refs):
            TB = x_ref.shape[0]
            acc = jnp.zeros((TB, 2 * K), jnp.float32)
            for s in range(S):
                xs = x_ref[:, 128 * s : 128 * (s + 1)]
                h = jnp.maximum(
                    jax.lax.dot_general(
                        xs, w1_ref[...], (((1,), (0,)), ((), ())),
                        preferred_element_type=jnp.float32,
                    )
                    + b1_ref[...],
                    0.0,
                )
                acc = acc + jax.lax.dot_general(
                    h, wr_ref[s], (((1,), (0,)), ((), ())),
                    preferred_element_type=jnp.float32,
                )
            delta = (acc[:, K:] + b2_ref[0, 1]) - (acc[:, :K] + b2_ref[0, 0])
            rowmax = jnp.max(delta, axis=1, keepdims=True)
            lane = jax.lax.broadcasted_iota(jnp.int32, delta.shape, 1)
            dec = jnp.where(
                (delta > 0.0) | ((lane == 0) & (rowmax <= 0.0)), 1.0, 0.0
            )
            dec_ref[q * TB : (q + 1) * TB, :] = dec
            keep_ref[q * TB : (q + 1) * TB, :] = jax.nn.sigmoid(delta)

    return _body


def kernel(slots, W1, b1, W2, b2):
    B, K, D = slots.shape
    F = W1.shape[1]
    S = K // 2
    x = jnp.reshape(slots, (B, K * D))
    z = jnp.zeros((D, F), jnp.float32)
    w1bd = jnp.block([[W1, z], [z, W1]])
    b1bd = jnp.concatenate([b1, b1]).reshape(1, 2 * F)
    cols = jax.lax.broadcasted_iota(jnp.int32, (S, 2 * F, 2 * K), 2)
    rows = jax.lax.broadcasted_iota(jnp.int32, (S, 2 * F, 2 * K), 1)
    sidx = jax.lax.broadcasted_iota(jnp.int32, (S, 2 * F, 2 * K), 0)
    cmod = cols % K
    v0 = jnp.concatenate([W2[:, 0], W2[:, 0]]).reshape(1, 2 * F, 1)
    v1 = jnp.concatenate([W2[:, 1], W2[:, 1]]).reshape(1, 2 * F, 1)
    vals = jnp.where(cols < K, v0, v1)
    wr = jnp.where(
        (cmod == 2 * sidx + (rows >= F).astype(jnp.int32))
        & ((rows < F) == (cmod == 2 * sidx)),
        vals,
        0.0,
    ).astype(jnp.float32)
    b2r = b2.reshape(1, 2)

    if B % (_NQ * _TBQ) == 0:
        nq, tbq = _NQ, _TBQ
    else:
        nq, tbq = 1, min(_TBQ, B)
    TB = nq * tbq
    grid = (B // TB,)
    x_specs = [
        pl.BlockSpec((tbq, K * D), lambda i, q=q: (nq * i + q, 0))
        for q in range(nq)
    ]
    dec, keep = pl.pallas_call(
        _make_body(nq),
        grid=grid,
        in_specs=x_specs
        + [
            pl.BlockSpec((2 * D, 2 * F), lambda i: (0, 0)),
            pl.BlockSpec((1, 2 * F), lambda i: (0, 0)),
            pl.BlockSpec((S, 2 * F, 2 * K), lambda i: (0, 0, 0)),
            pl.BlockSpec(memory_space=pltpu.SMEM),
        ],
        out_specs=[
            pl.BlockSpec((TB, K), lambda i: (i, 0)),
            pl.BlockSpec((TB, K), lambda i: (i, 0)),
        ],
        out_shape=[
            jax.ShapeDtypeStruct((B, K), jnp.float32),
            jax.ShapeDtypeStruct((B, K), jnp.float32),
        ],
        compiler_params=pltpu.CompilerParams(
            dimension_semantics=("parallel",),
        ),
    )(*([x] * nq), w1bd, b1bd, wr, b2r)
    return (dec, keep)
